# baseline (speedup 1.0000x reference)
import functools

import jax
import jax.numpy as jnp
import numpy as np
from jax import lax
from jax.experimental import pallas as pl
from jax.experimental.pallas import tpu as pltpu
from jax.experimental.pallas import tpu_sc as plsc

_SC_CORES = 2
_SC_SUBCORES = 16
_GATHER_CHUNK = 512
_L_PE = 6


def _sc_gather(emb, idx_flat):
    nk = idx_flat.shape[0]
    _, de = emb.shape
    nw = _SC_CORES * _SC_SUBCORES
    b_per_w = nk // nw
    ch = min(_GATHER_CHUNK, b_per_w)
    mesh = plsc.VectorSubcoreMesh(core_axis_name="c", subcore_axis_name="s")

    @functools.partial(
        pl.kernel,
        mesh=mesh,
        out_type=jax.ShapeDtypeStruct((nk, de), jnp.float32),
        scratch_types=[
            pltpu.VMEM((ch,), jnp.int32),
            pltpu.VMEM((ch, de), jnp.float32),
            pltpu.SemaphoreType.DMA,
        ],
    )
    def gather_kernel(emb_hbm, idx_hbm, out_hbm, idx_v, rows_v, sem):
        wid = lax.axis_index("s") * _SC_CORES + lax.axis_index("c")
        base = wid * b_per_w

        @pl.loop(0, b_per_w, step=ch)
        def _(off):
            pltpu.sync_copy(idx_hbm.at[pl.ds(base + off, ch)], idx_v)
            pltpu.async_copy(emb_hbm.at[idx_v], rows_v, sem).wait()
            pltpu.sync_copy(rows_v, out_hbm.at[pl.ds(base + off, ch)])

    return gather_kernel(emb, idx_flat)


def _mlp_kernel(rs_ref, rd_ref, dep_ref, feat_ref,
                w1_ref, b1_ref, w2_ref, b2_ref, wsig_ref, bsig_ref,
                wt1h_ref, wt1d_ref, bt1_ref, wt2_ref, bt2_ref,
                sig_ref, tex_ref):
    f32 = jnp.float32
    rd = rd_ref[...]
    pts = rs_ref[...] + rd * dep_ref[...]

    j = lax.broadcasted_iota(jnp.int32, (1, 2 * _L_PE), 1)
    fr12 = jnp.exp2((j % _L_PE).astype(jnp.float32)) * np.float32(np.pi)
    ph12 = jnp.where(j >= _L_PE, np.float32(0.5 * np.pi), np.float32(0.0))
    pe_parts = [jnp.sin(pts[:, c:c + 1] * fr12 + ph12) for c in range(3)]
    pe = jnp.concatenate(pe_parts, axis=1)

    de = w1_ref.shape[0] - 39
    acc = jnp.dot(pts, w1_ref[0:3, :], preferred_element_type=f32)
    acc += jnp.dot(pe, w1_ref[3:39, :], preferred_element_type=f32)
    acc += jnp.dot(feat_ref[:, 0:de], w1_ref[39:39 + de, :], preferred_element_type=f32)
    h1 = jnp.maximum(acc + b1_ref[...], 0.0)
    h2 = jnp.maximum(jnp.dot(h1, w2_ref[...], preferred_element_type=f32) + b2_ref[...], 0.0)
    sig_ref[...] = jnp.dot(h2, wsig_ref[...], preferred_element_type=f32) + bsig_ref[...]
    t = jnp.dot(h2, wt1h_ref[...], preferred_element_type=f32)
    t += jnp.dot(rd, wt1d_ref[...], preferred_element_type=f32)
    t = jnp.maximum(t + bt1_ref[...], 0.0)
    tex_ref[...] = jax.nn.sigmoid(jnp.dot(t, wt2_ref[...], preferred_element_type=f32) + bt2_ref[...])


def _mlp_call(rs_flat, rd_flat, dep_flat, feat,
              w1, b1, w2, b2, wsig, bsig, wt1h, wt1d, bt1, wt2, bt2, p_blk):
    nk = rs_flat.shape[0]
    grid = nk // p_blk
    row = lambda i: (i, 0)
    rep = lambda i: (0, 0)
    f32 = jnp.float32
    return pl.pallas_call(
        _mlp_kernel,
        grid=(grid,),
        in_specs=[
            pl.BlockSpec((p_blk, 3), row),
            pl.BlockSpec((p_blk, 3), row),
            pl.BlockSpec((p_blk, 1), row),
            pl.BlockSpec((p_blk, feat.shape[1]), row),
            pl.BlockSpec(w1.shape, rep),
            pl.BlockSpec(b1.shape, rep),
            pl.BlockSpec(w2.shape, rep),
            pl.BlockSpec(b2.shape, rep),
            pl.BlockSpec(wsig.shape, rep),
            pl.BlockSpec(bsig.shape, rep),
            pl.BlockSpec(wt1h.shape, rep),
            pl.BlockSpec(wt1d.shape, rep),
            pl.BlockSpec(bt1.shape, rep),
            pl.BlockSpec(wt2.shape, rep),
            pl.BlockSpec(bt2.shape, rep),
        ],
        out_specs=[
            pl.BlockSpec((p_blk, 1), row),
            pl.BlockSpec((p_blk, 3), row),
        ],
        out_shape=[
            jax.ShapeDtypeStruct((nk, 1), f32),
            jax.ShapeDtypeStruct((nk, 3), f32),
        ],
    )(rs_flat, rd_flat, dep_flat, feat, w1, b1, w2, b2, wsig, bsig,
      wt1h, wt1d, bt1, wt2, bt2)


def _render_kernel(sig_ref, t0_ref, t1_ref, t2_ref, dep_ref, dst_ref, idx_ref,
                   probs_ref, depth_ref, miss_ref, col_ref):
    k = sig_ref.shape[1]
    maskf = (idx_ref[...] != -1).astype(jnp.float32)
    fe = jnp.maximum(sig_ref[...], 0.0) * dst_ref[...] * maskf
    row = lax.broadcasted_iota(jnp.int32, (k, k), 0)
    col = lax.broadcasted_iota(jnp.int32, (k, k), 1)
    upper = (row < col).astype(jnp.float32)
    cum = jnp.dot(fe, upper, precision=lax.Precision.HIGHEST,
                  preferred_element_type=jnp.float32)
    probs = (1.0 - jnp.exp(-fe)) * jnp.exp(-cum)
    probs_ref[...] = probs
    depth_ref[...] = jnp.sum(dep_ref[...] * probs, axis=1, keepdims=True)
    miss_ref[...] = 1.0 - jnp.sum(probs, axis=1, keepdims=True)
    c0 = jnp.sum(t0_ref[...] * probs, axis=1, keepdims=True)
    c1 = jnp.sum(t1_ref[...] * probs, axis=1, keepdims=True)
    c2 = jnp.sum(t2_ref[...] * probs, axis=1, keepdims=True)
    col_ref[...] = jnp.concatenate([c0, c1, c2], axis=1)


def _render_call(sigma2, tex0, tex1, tex2, sampled_depth, sampled_dists, sampled_idx, r_blk):
    n, k = sigma2.shape
    grid = n // r_blk
    row = lambda i: (i, 0)
    f32 = jnp.float32
    return pl.pallas_call(
        _render_kernel,
        grid=(grid,),
        in_specs=[pl.BlockSpec((r_blk, k), row)] * 7,
        out_specs=[
            pl.BlockSpec((r_blk, k), row),
            pl.BlockSpec((r_blk, 1), row),
            pl.BlockSpec((r_blk, 1), row),
            pl.BlockSpec((r_blk, 3), row),
        ],
        out_shape=[
            jax.ShapeDtypeStruct((n, k), f32),
            jax.ShapeDtypeStruct((n, 1), f32),
            jax.ShapeDtypeStruct((n, 1), f32),
            jax.ShapeDtypeStruct((n, 3), f32),
        ],
    )(sigma2, tex0, tex1, tex2, sampled_depth, sampled_dists, sampled_idx)


def kernel(ray_start, ray_dir, sampled_depth, sampled_idx, sampled_dists, emb,
           W1, b1, W2, b2, Wsig, bsig, Wt1, bt1, Wt2, bt2):
    n, k = sampled_depth.shape
    nk = n * k
    hid = W2.shape[0]

    idx_flat = jnp.maximum(sampled_idx.reshape(nk), 0).astype(jnp.int32)
    emb_pad = jnp.pad(emb, ((0, 0), (0, 128 - emb.shape[1])))
    feat = _sc_gather(emb_pad, idx_flat)

    rs_flat = jnp.broadcast_to(ray_start[:, None, :], (n, k, 3)).reshape(nk, 3)
    rd_flat = jnp.broadcast_to(ray_dir[:, None, :], (n, k, 3)).reshape(nk, 3)
    dep_flat = sampled_depth.reshape(nk, 1)

    sig, tex = _mlp_call(
        rs_flat, rd_flat, dep_flat, feat,
        W1, b1.reshape(1, -1), W2, b2.reshape(1, -1), Wsig, bsig.reshape(1, 1),
        Wt1[:hid], Wt1[hid:], bt1.reshape(1, -1), Wt2, bt2.reshape(1, -1),
        p_blk=4096)

    sigma2 = sig.reshape(n, k)
    tex3 = tex.reshape(n, k, 3)
    probs, depths, missed, colors = _render_call(
        sigma2, tex3[..., 0], tex3[..., 1], tex3[..., 2],
        sampled_depth, sampled_dists, sampled_idx, r_blk=512)
    return probs, depths.reshape(n), missed.reshape(n), colors


# speedup vs baseline: 1.0001x; 1.0001x over previous
import functools

import jax
import jax.numpy as jnp
import numpy as np
from jax import lax
from jax.experimental import pallas as pl
from jax.experimental.pallas import tpu as pltpu
from jax.experimental.pallas import tpu_sc as plsc

_SC_CORES = 2
_SC_SUBCORES = 16
_GATHER_CHUNK = 128
_L_PE = 6


def _sc_gather(emb, idx_flat):
    nk = idx_flat.shape[0]
    _, de = emb.shape
    w = _GATHER_CHUNK
    mesh = plsc.VectorSubcoreMesh(core_axis_name="c", subcore_axis_name="s")
    idx2 = idx_flat.reshape(1, nk)

    @functools.partial(
        pl.kernel,
        mesh=mesh,
        out_type=jax.ShapeDtypeStruct((nk, de), jnp.float32),
    )
    def gather_kernel(emb_hbm, idx_hbm, out_hbm):
        def body(i_vmem, o_vmem):
            pltpu.sync_copy(emb_hbm.at[i_vmem.at[0]], o_vmem)

        pltpu.emit_pipeline(
            body,
            grid=(nk // w,),
            in_specs=[pl.BlockSpec((1, w), lambda i: (0, i))],
            out_specs=[pl.BlockSpec((w, de), lambda i: (i, 0))],
            core_axis_name=("c", "s"),
            dimension_semantics=(pltpu.PARALLEL,),
        )(idx_hbm, out_hbm)

    return gather_kernel(emb, idx2)


def _mlp_kernel(rs_ref, rd_ref, dep_ref, feat_ref,
                w1_ref, b1_ref, w2_ref, b2_ref, wsig_ref, bsig_ref,
                wt1h_ref, wt1d_ref, bt1_ref, wt2_ref, bt2_ref,
                sig_ref, tex_ref):
    f32 = jnp.float32
    rd = rd_ref[...]
    pts = rs_ref[...] + rd * dep_ref[...]

    j = lax.broadcasted_iota(jnp.int32, (1, 2 * _L_PE), 1)
    fr12 = jnp.exp2((j % _L_PE).astype(jnp.float32)) * np.float32(np.pi)
    ph12 = jnp.where(j >= _L_PE, np.float32(0.5 * np.pi), np.float32(0.0))
    pe_parts = [jnp.sin(pts[:, c:c + 1] * fr12 + ph12) for c in range(3)]
    pe = jnp.concatenate(pe_parts, axis=1)

    de = w1_ref.shape[0] - 39
    acc = jnp.dot(pts, w1_ref[0:3, :], preferred_element_type=f32)
    acc += jnp.dot(pe, w1_ref[3:39, :], preferred_element_type=f32)
    acc += jnp.dot(feat_ref[:, 0:de], w1_ref[39:39 + de, :], preferred_element_type=f32)
    h1 = jnp.maximum(acc + b1_ref[...], 0.0)
    h2 = jnp.maximum(jnp.dot(h1, w2_ref[...], preferred_element_type=f32) + b2_ref[...], 0.0)
    sig_ref[...] = jnp.dot(h2, wsig_ref[...], preferred_element_type=f32) + bsig_ref[...]
    t = jnp.dot(h2, wt1h_ref[...], preferred_element_type=f32)
    t += jnp.dot(rd, wt1d_ref[...], preferred_element_type=f32)
    t = jnp.maximum(t + bt1_ref[...], 0.0)
    tex_ref[...] = jax.nn.sigmoid(jnp.dot(t, wt2_ref[...], preferred_element_type=f32) + bt2_ref[...])


def _mlp_call(rs_flat, rd_flat, dep_flat, feat,
              w1, b1, w2, b2, wsig, bsig, wt1h, wt1d, bt1, wt2, bt2, p_blk):
    nk = rs_flat.shape[0]
    grid = nk // p_blk
    row = lambda i: (i, 0)
    rep = lambda i: (0, 0)
    f32 = jnp.float32
    return pl.pallas_call(
        _mlp_kernel,
        grid=(grid,),
        in_specs=[
            pl.BlockSpec((p_blk, 3), row),
            pl.BlockSpec((p_blk, 3), row),
            pl.BlockSpec((p_blk, 1), row),
            pl.BlockSpec((p_blk, feat.shape[1]), row),
            pl.BlockSpec(w1.shape, rep),
            pl.BlockSpec(b1.shape, rep),
            pl.BlockSpec(w2.shape, rep),
            pl.BlockSpec(b2.shape, rep),
            pl.BlockSpec(wsig.shape, rep),
            pl.BlockSpec(bsig.shape, rep),
            pl.BlockSpec(wt1h.shape, rep),
            pl.BlockSpec(wt1d.shape, rep),
            pl.BlockSpec(bt1.shape, rep),
            pl.BlockSpec(wt2.shape, rep),
            pl.BlockSpec(bt2.shape, rep),
        ],
        out_specs=[
            pl.BlockSpec((p_blk, 1), row),
            pl.BlockSpec((p_blk, 3), row),
        ],
        out_shape=[
            jax.ShapeDtypeStruct((nk, 1), f32),
            jax.ShapeDtypeStruct((nk, 3), f32),
        ],
    )(rs_flat, rd_flat, dep_flat, feat, w1, b1, w2, b2, wsig, bsig,
      wt1h, wt1d, bt1, wt2, bt2)


def _render_kernel(sig_ref, t0_ref, t1_ref, t2_ref, dep_ref, dst_ref, idx_ref,
                   probs_ref, depth_ref, miss_ref, col_ref):
    k = sig_ref.shape[1]
    maskf = (idx_ref[...] != -1).astype(jnp.float32)
    fe = jnp.maximum(sig_ref[...], 0.0) * dst_ref[...] * maskf
    row = lax.broadcasted_iota(jnp.int32, (k, k), 0)
    col = lax.broadcasted_iota(jnp.int32, (k, k), 1)
    upper = (row < col).astype(jnp.float32)
    cum = jnp.dot(fe, upper, precision=lax.Precision.HIGHEST,
                  preferred_element_type=jnp.float32)
    probs = (1.0 - jnp.exp(-fe)) * jnp.exp(-cum)
    probs_ref[...] = probs
    depth_ref[...] = jnp.sum(dep_ref[...] * probs, axis=1, keepdims=True)
    miss_ref[...] = 1.0 - jnp.sum(probs, axis=1, keepdims=True)
    c0 = jnp.sum(t0_ref[...] * probs, axis=1, keepdims=True)
    c1 = jnp.sum(t1_ref[...] * probs, axis=1, keepdims=True)
    c2 = jnp.sum(t2_ref[...] * probs, axis=1, keepdims=True)
    col_ref[...] = jnp.concatenate([c0, c1, c2], axis=1)


def _render_call(sigma2, tex0, tex1, tex2, sampled_depth, sampled_dists, sampled_idx, r_blk):
    n, k = sigma2.shape
    grid = n // r_blk
    row = lambda i: (i, 0)
    f32 = jnp.float32
    return pl.pallas_call(
        _render_kernel,
        grid=(grid,),
        in_specs=[pl.BlockSpec((r_blk, k), row)] * 7,
        out_specs=[
            pl.BlockSpec((r_blk, k), row),
            pl.BlockSpec((r_blk, 1), row),
            pl.BlockSpec((r_blk, 1), row),
            pl.BlockSpec((r_blk, 3), row),
        ],
        out_shape=[
            jax.ShapeDtypeStruct((n, k), f32),
            jax.ShapeDtypeStruct((n, 1), f32),
            jax.ShapeDtypeStruct((n, 1), f32),
            jax.ShapeDtypeStruct((n, 3), f32),
        ],
    )(sigma2, tex0, tex1, tex2, sampled_depth, sampled_dists, sampled_idx)


def kernel(ray_start, ray_dir, sampled_depth, sampled_idx, sampled_dists, emb,
           W1, b1, W2, b2, Wsig, bsig, Wt1, bt1, Wt2, bt2):
    n, k = sampled_depth.shape
    nk = n * k
    hid = W2.shape[0]

    idx_flat = jnp.maximum(sampled_idx.reshape(nk), 0).astype(jnp.int32)
    emb_pad = jnp.pad(emb, ((0, 0), (0, 128 - emb.shape[1])))
    feat = _sc_gather(emb_pad, idx_flat)

    rs_flat = jnp.broadcast_to(ray_start[:, None, :], (n, k, 3)).reshape(nk, 3)
    rd_flat = jnp.broadcast_to(ray_dir[:, None, :], (n, k, 3)).reshape(nk, 3)
    dep_flat = sampled_depth.reshape(nk, 1)

    sig, tex = _mlp_call(
        rs_flat, rd_flat, dep_flat, feat,
        W1, b1.reshape(1, -1), W2, b2.reshape(1, -1), Wsig, bsig.reshape(1, 1),
        Wt1[:hid], Wt1[hid:], bt1.reshape(1, -1), Wt2, bt2.reshape(1, -1),
        p_blk=4096)

    sigma2 = sig.reshape(n, k)
    tex3 = tex.reshape(n, k, 3)
    probs, depths, missed, colors = _render_call(
        sigma2, tex3[..., 0], tex3[..., 1], tex3[..., 2],
        sampled_depth, sampled_dists, sampled_idx, r_blk=512)
    return probs, depths.reshape(n), missed.reshape(n), colors


# speedup vs baseline: 1.9606x; 1.9605x over previous
import functools

import jax
import jax.numpy as jnp
import numpy as np
from jax import lax
from jax.experimental import pallas as pl
from jax.experimental.pallas import tpu as pltpu
from jax.experimental.pallas import tpu_sc as plsc

_SC_CORES = 2
_SC_SUBCORES = 16
_GATHER_CHUNK = 512
_L_PE = 6


def _sc_gather(emb3, g_flat):
    nk = g_flat.shape[0]
    vq, de = emb3.shape
    w = _GATHER_CHUNK
    mesh = plsc.VectorSubcoreMesh(core_axis_name="c", subcore_axis_name="s")

    rows_per_tile = vq // _SC_SUBCORES
    stage_rows = 56
    nw = _SC_CORES * _SC_SUBCORES
    b_per_w = nk // nw

    @functools.partial(
        pl.kernel,
        mesh=mesh,
        out_type=jax.ShapeDtypeStruct((nk, de), jnp.int32),
        scratch_types=[
            pltpu.VMEM_SHARED((vq, de), jnp.int32),
            pltpu.VMEM((stage_rows, de), jnp.int32),
            pltpu.VMEM((w,), jnp.int32),
            pltpu.VMEM((w, de), jnp.int32),
            pltpu.SemaphoreType.DMA,
        ],
    )
    def gather_kernel(emb_hbm, idx_hbm, out_hbm, sp_table, stage_v, idx_v, rows_v, sem):
        sbase = lax.axis_index("s") * rows_per_tile

        @pl.loop(0, rows_per_tile, step=stage_rows)
        def _(off):
            pltpu.sync_copy(emb_hbm.at[pl.ds(sbase + off, stage_rows)], stage_v)
            pltpu.sync_copy(stage_v, sp_table.at[pl.ds(sbase + off, stage_rows)])

        plsc.subcore_barrier()

        wid = lax.axis_index("s") * _SC_CORES + lax.axis_index("c")
        base = wid * b_per_w

        @pl.loop(0, b_per_w, step=w)
        def _(off):
            pltpu.sync_copy(idx_hbm.at[pl.ds(base + off, w)], idx_v)
            pltpu.async_copy(sp_table.at[idx_v], rows_v, sem).wait()
            pltpu.sync_copy(rows_v, out_hbm.at[pl.ds(base + off, w)])

    return gather_kernel(emb3, g_flat)


def _mlp_kernel(rs_ref, rd_ref, dep_ref, feat_ref, q_ref,
                w1_ref, w1c4_ref, b1_ref, w2_ref, b2_ref, wsig_ref, bsig_ref,
                wt1h_ref, wt1d_ref, bt1_ref, wt2_ref, bt2_ref,
                sig_ref, tex_ref):
    f32 = jnp.float32
    rd = rd_ref[...]
    pts = rs_ref[...] + rd * dep_ref[...]

    j = lax.broadcasted_iota(jnp.int32, (1, 2 * _L_PE), 1)
    fr12 = jnp.exp2((j % _L_PE).astype(jnp.float32)) * np.float32(np.pi)
    ph12 = jnp.where(j >= _L_PE, np.float32(0.5 * np.pi), np.float32(0.0))
    pe_parts = [jnp.sin(pts[:, c:c + 1] * fr12 + ph12) for c in range(3)]
    pe = jnp.concatenate(pe_parts, axis=1)

    p = feat_ref.shape[0]
    lane = lax.broadcasted_iota(jnp.int32, (p, feat_ref.shape[1]), 1)
    qmask = (lane // (feat_ref.shape[1] // 8) == q_ref[...]).astype(f32)
    featm = feat_ref[...].astype(f32) * qmask
    acc = jnp.dot(pts, w1_ref[0:3, :], preferred_element_type=f32)
    acc += jnp.dot(pe, w1_ref[3:39, :], preferred_element_type=f32)
    acc += jnp.dot(featm, w1c4_ref[...], preferred_element_type=f32)
    h1 = jnp.maximum(acc + b1_ref[...], 0.0)
    h2 = jnp.maximum(jnp.dot(h1, w2_ref[...], preferred_element_type=f32) + b2_ref[...], 0.0)
    sig_ref[...] = jnp.dot(h2, wsig_ref[...], preferred_element_type=f32) + bsig_ref[...]
    t = jnp.dot(h2, wt1h_ref[...], preferred_element_type=f32)
    t += jnp.dot(rd, wt1d_ref[...], preferred_element_type=f32)
    t = jnp.maximum(t + bt1_ref[...], 0.0)
    tex_ref[...] = jax.nn.sigmoid(jnp.dot(t, wt2_ref[...], preferred_element_type=f32) + bt2_ref[...])


def _mlp_call(rs_flat, rd_flat, dep_flat, feat, q_flat,
              w1, w1c4, b1, w2, b2, wsig, bsig, wt1h, wt1d, bt1, wt2, bt2, p_blk):
    nk = rs_flat.shape[0]
    grid = nk // p_blk
    row = lambda i: (i, 0)
    rep = lambda i: (0, 0)
    f32 = jnp.float32
    return pl.pallas_call(
        _mlp_kernel,
        grid=(grid,),
        in_specs=[
            pl.BlockSpec((p_blk, 3), row),
            pl.BlockSpec((p_blk, 3), row),
            pl.BlockSpec((p_blk, 1), row),
            pl.BlockSpec((p_blk, feat.shape[1]), row),
            pl.BlockSpec((p_blk, 1), row),
            pl.BlockSpec(w1.shape, rep),
            pl.BlockSpec(w1c4.shape, rep),
            pl.BlockSpec(b1.shape, rep),
            pl.BlockSpec(w2.shape, rep),
            pl.BlockSpec(b2.shape, rep),
            pl.BlockSpec(wsig.shape, rep),
            pl.BlockSpec(bsig.shape, rep),
            pl.BlockSpec(wt1h.shape, rep),
            pl.BlockSpec(wt1d.shape, rep),
            pl.BlockSpec(bt1.shape, rep),
            pl.BlockSpec(wt2.shape, rep),
            pl.BlockSpec(bt2.shape, rep),
        ],
        out_specs=[
            pl.BlockSpec((p_blk, 1), row),
            pl.BlockSpec((p_blk, 3), row),
        ],
        out_shape=[
            jax.ShapeDtypeStruct((nk, 1), f32),
            jax.ShapeDtypeStruct((nk, 3), f32),
        ],
    )(rs_flat, rd_flat, dep_flat, feat, q_flat, w1, w1c4, b1, w2, b2, wsig, bsig,
      wt1h, wt1d, bt1, wt2, bt2)


def _render_kernel(sig_ref, t0_ref, t1_ref, t2_ref, dep_ref, dst_ref, idx_ref,
                   probs_ref, depth_ref, miss_ref, col_ref):
    k = sig_ref.shape[1]
    maskf = (idx_ref[...] != -1).astype(jnp.float32)
    fe = jnp.maximum(sig_ref[...], 0.0) * dst_ref[...] * maskf
    row = lax.broadcasted_iota(jnp.int32, (k, k), 0)
    col = lax.broadcasted_iota(jnp.int32, (k, k), 1)
    upper = (row < col).astype(jnp.float32)
    cum = jnp.dot(fe, upper, precision=lax.Precision.HIGHEST,
                  preferred_element_type=jnp.float32)
    probs = (1.0 - jnp.exp(-fe)) * jnp.exp(-cum)
    probs_ref[...] = probs
    depth_ref[...] = jnp.sum(dep_ref[...] * probs, axis=1, keepdims=True)
    miss_ref[...] = 1.0 - jnp.sum(probs, axis=1, keepdims=True)
    c0 = jnp.sum(t0_ref[...] * probs, axis=1, keepdims=True)
    c1 = jnp.sum(t1_ref[...] * probs, axis=1, keepdims=True)
    c2 = jnp.sum(t2_ref[...] * probs, axis=1, keepdims=True)
    col_ref[...] = jnp.concatenate([c0, c1, c2], axis=1)


def _render_call(sigma2, tex0, tex1, tex2, sampled_depth, sampled_dists, sampled_idx, r_blk):
    n, k = sigma2.shape
    grid = n // r_blk
    row = lambda i: (i, 0)
    f32 = jnp.float32
    return pl.pallas_call(
        _render_kernel,
        grid=(grid,),
        in_specs=[pl.BlockSpec((r_blk, k), row)] * 7,
        out_specs=[
            pl.BlockSpec((r_blk, k), row),
            pl.BlockSpec((r_blk, 1), row),
            pl.BlockSpec((r_blk, 1), row),
            pl.BlockSpec((r_blk, 3), row),
        ],
        out_shape=[
            jax.ShapeDtypeStruct((n, k), f32),
            jax.ShapeDtypeStruct((n, 1), f32),
            jax.ShapeDtypeStruct((n, 1), f32),
            jax.ShapeDtypeStruct((n, 3), f32),
        ],
    )(sigma2, tex0, tex1, tex2, sampled_depth, sampled_dists, sampled_idx)


def kernel(ray_start, ray_dir, sampled_depth, sampled_idx, sampled_dists, emb,
           W1, b1, W2, b2, Wsig, bsig, Wt1, bt1, Wt2, bt2):
    n, k = sampled_depth.shape
    nk = n * k
    hid = W2.shape[0]

    idx_flat = jnp.maximum(sampled_idx.reshape(nk), 0).astype(jnp.int32)
    v, de0 = emb.shape
    per = 256 // de0
    unit = _SC_SUBCORES * 56
    vq = -(-(v // per) // unit) * unit
    emb3 = jnp.pad(emb.astype(jnp.bfloat16), ((0, vq * per - v), (0, 0)))
    emb3 = lax.bitcast_convert_type(emb3.reshape(vq, 128, 2), jnp.int32)
    g_flat = idx_flat // per
    q_flat = (idx_flat % per).reshape(nk, 1)
    n_stripe = 8
    stripe = nk // n_stripe
    feat_i32 = jnp.concatenate(
        [_sc_gather(emb3, lax.dynamic_slice_in_dim(g_flat, s * stripe, stripe))
         for s in range(n_stripe)], axis=0)
    feat = lax.bitcast_convert_type(feat_i32, jnp.bfloat16).reshape(nk, 256)

    rs_flat = jnp.broadcast_to(ray_start[:, None, :], (n, k, 3)).reshape(nk, 3)
    rd_flat = jnp.broadcast_to(ray_dir[:, None, :], (n, k, 3)).reshape(nk, 3)
    dep_flat = sampled_depth.reshape(nk, 1)

    w1c4 = jnp.concatenate([W1[39:]] * per, axis=0)
    sig, tex = _mlp_call(
        rs_flat, rd_flat, dep_flat, feat, q_flat,
        W1[:39], w1c4, b1.reshape(1, -1), W2, b2.reshape(1, -1), Wsig, bsig.reshape(1, 1),
        Wt1[:hid], Wt1[hid:], bt1.reshape(1, -1), Wt2, bt2.reshape(1, -1),
        p_blk=4096)

    sigma2 = sig.reshape(n, k)
    tex3 = tex.reshape(n, k, 3)
    probs, depths, missed, colors = _render_call(
        sigma2, tex3[..., 0], tex3[..., 1], tex3[..., 2],
        sampled_depth, sampled_dists, sampled_idx, r_blk=512)
    return probs, depths.reshape(n), missed.reshape(n), colors


# speedup vs baseline: 2.8708x; 1.4643x over previous
import functools

import jax
import jax.numpy as jnp
import numpy as np
from jax import lax
from jax.experimental import pallas as pl
from jax.experimental.pallas import tpu as pltpu
from jax.experimental.pallas import tpu_sc as plsc

_SC_CORES = 2
_SC_SUBCORES = 16
_GATHER_CHUNK = 512
_L_PE = 6


def _sc_gather(emb3, g_flat):
    nk = g_flat.shape[0]
    vq, de = emb3.shape
    w = _GATHER_CHUNK
    mesh = plsc.VectorSubcoreMesh(core_axis_name="c", subcore_axis_name="s")

    rows_per_tile = vq // _SC_SUBCORES
    stage_rows = 56
    nw = _SC_CORES * _SC_SUBCORES
    b_per_w = nk // nw

    @functools.partial(
        pl.kernel,
        mesh=mesh,
        out_type=jax.ShapeDtypeStruct((nk, de), jnp.int32),
        scratch_types=[
            pltpu.VMEM_SHARED((vq, de), jnp.int32),
            pltpu.VMEM((stage_rows, de), jnp.int32),
            pltpu.VMEM((w,), jnp.int32),
            pltpu.VMEM((w, de), jnp.int32),
            pltpu.SemaphoreType.DMA,
        ],
    )
    def gather_kernel(emb_hbm, idx_hbm, out_hbm, sp_table, stage_v, idx_v, rows_v, sem):
        sbase = lax.axis_index("s") * rows_per_tile

        @pl.loop(0, rows_per_tile, step=stage_rows)
        def _(off):
            pltpu.sync_copy(emb_hbm.at[pl.ds(sbase + off, stage_rows)], stage_v)
            pltpu.sync_copy(stage_v, sp_table.at[pl.ds(sbase + off, stage_rows)])

        plsc.subcore_barrier()

        wid = lax.axis_index("s") * _SC_CORES + lax.axis_index("c")
        base = wid * b_per_w

        @pl.loop(0, b_per_w, step=w)
        def _(off):
            pltpu.sync_copy(idx_hbm.at[pl.ds(base + off, w)], idx_v)
            pltpu.async_copy(sp_table.at[idx_v], rows_v, sem).wait()
            pltpu.sync_copy(rows_v, out_hbm.at[pl.ds(base + off, w)])

    return gather_kernel(emb3, g_flat)


def _mlp_kernel(rs_ref, rd_ref, dep_ref, feat_ref, q_ref,
                w1_ref, w1c4_ref, b1_ref, w2_ref, b2_ref, wsig_ref, bsig_ref,
                wt1h_ref, wt1d_ref, bt1_ref, wt2_ref, bt2_ref,
                sig_ref, tex_ref):
    f32 = jnp.float32
    rd = rd_ref[...]
    pts = rs_ref[...] + rd * dep_ref[...]

    j = lax.broadcasted_iota(jnp.int32, (1, 2 * _L_PE), 1)
    fr12 = jnp.exp2((j % _L_PE).astype(jnp.float32)) * np.float32(np.pi)
    ph12 = jnp.where(j >= _L_PE, np.float32(0.5 * np.pi), np.float32(0.0))
    pe_parts = [jnp.sin(pts[:, c:c + 1] * fr12 + ph12) for c in range(3)]
    pe = jnp.concatenate(pe_parts, axis=1)

    bf = jnp.bfloat16
    p = feat_ref.shape[0]
    fi = feat_ref[...]
    f_lo = lax.bitcast_convert_type(fi << 16, f32)
    f_hi = lax.bitcast_convert_type(fi & jnp.int32(-65536), f32)
    lane = lax.broadcasted_iota(jnp.int32, (p, 128), 1)
    q = q_ref[...]
    featm = jnp.where(lane // 32 == q, f_lo, 0.0) + jnp.where(lane // 32 == q - 4, f_hi, 0.0)
    acc = jnp.dot(pts.astype(bf), w1_ref[0:3, :], preferred_element_type=f32)
    acc += jnp.dot(pe.astype(bf), w1_ref[3:39, :], preferred_element_type=f32)
    acc += jnp.dot(featm.astype(bf), w1c4_ref[...], preferred_element_type=f32)
    h1 = jnp.maximum(acc + b1_ref[...], 0.0).astype(bf)
    h2 = jnp.maximum(jnp.dot(h1, w2_ref[...], preferred_element_type=f32) + b2_ref[...], 0.0).astype(bf)
    sig_ref[...] = jnp.dot(h2, wsig_ref[...], preferred_element_type=f32) + bsig_ref[...]
    t = jnp.dot(h2, wt1h_ref[...], preferred_element_type=f32)
    t += jnp.dot(rd.astype(bf), wt1d_ref[...], preferred_element_type=f32)
    t = jnp.maximum(t + bt1_ref[...], 0.0).astype(bf)
    tex_ref[...] = jax.nn.sigmoid(jnp.dot(t, wt2_ref[...], preferred_element_type=f32) + bt2_ref[...])


def _mlp_call(rs_flat, rd_flat, dep_flat, feat, q_flat,
              w1, w1c4, b1, w2, b2, wsig, bsig, wt1h, wt1d, bt1, wt2, bt2, p_blk):
    nk = rs_flat.shape[0]
    grid = nk // p_blk
    row = lambda i: (i, 0)
    rep = lambda i: (0, 0)
    f32 = jnp.float32
    return pl.pallas_call(
        _mlp_kernel,
        grid=(grid,),
        in_specs=[
            pl.BlockSpec((p_blk, 3), row),
            pl.BlockSpec((p_blk, 3), row),
            pl.BlockSpec((p_blk, 1), row),
            pl.BlockSpec((p_blk, feat.shape[1]), row),
            pl.BlockSpec((p_blk, 1), row),
            pl.BlockSpec(w1.shape, rep),
            pl.BlockSpec(w1c4.shape, rep),
            pl.BlockSpec(b1.shape, rep),
            pl.BlockSpec(w2.shape, rep),
            pl.BlockSpec(b2.shape, rep),
            pl.BlockSpec(wsig.shape, rep),
            pl.BlockSpec(bsig.shape, rep),
            pl.BlockSpec(wt1h.shape, rep),
            pl.BlockSpec(wt1d.shape, rep),
            pl.BlockSpec(bt1.shape, rep),
            pl.BlockSpec(wt2.shape, rep),
            pl.BlockSpec(bt2.shape, rep),
        ],
        out_specs=[
            pl.BlockSpec((p_blk, 1), row),
            pl.BlockSpec((p_blk, 3), row),
        ],
        out_shape=[
            jax.ShapeDtypeStruct((nk, 1), f32),
            jax.ShapeDtypeStruct((nk, 3), f32),
        ],
    )(rs_flat, rd_flat, dep_flat, feat, q_flat, w1, w1c4, b1, w2, b2, wsig, bsig,
      wt1h, wt1d, bt1, wt2, bt2)


def _render_kernel(sig_ref, t0_ref, t1_ref, t2_ref, dep_ref, dst_ref, idx_ref,
                   probs_ref, depth_ref, miss_ref, col_ref):
    k = sig_ref.shape[1]
    maskf = (idx_ref[...] != -1).astype(jnp.float32)
    fe = jnp.maximum(sig_ref[...], 0.0) * dst_ref[...] * maskf
    row = lax.broadcasted_iota(jnp.int32, (k, k), 0)
    col = lax.broadcasted_iota(jnp.int32, (k, k), 1)
    upper = (row < col).astype(jnp.float32)
    cum = jnp.dot(fe, upper, precision=lax.Precision.HIGHEST,
                  preferred_element_type=jnp.float32)
    probs = (1.0 - jnp.exp(-fe)) * jnp.exp(-cum)
    probs_ref[...] = probs
    depth_ref[...] = jnp.sum(dep_ref[...] * probs, axis=1, keepdims=True)
    miss_ref[...] = 1.0 - jnp.sum(probs, axis=1, keepdims=True)
    c0 = jnp.sum(t0_ref[...] * probs, axis=1, keepdims=True)
    c1 = jnp.sum(t1_ref[...] * probs, axis=1, keepdims=True)
    c2 = jnp.sum(t2_ref[...] * probs, axis=1, keepdims=True)
    col_ref[...] = jnp.concatenate([c0, c1, c2], axis=1)


def _render_call(sigma2, tex0, tex1, tex2, sampled_depth, sampled_dists, sampled_idx, r_blk):
    n, k = sigma2.shape
    grid = n // r_blk
    row = lambda i: (i, 0)
    f32 = jnp.float32
    return pl.pallas_call(
        _render_kernel,
        grid=(grid,),
        in_specs=[pl.BlockSpec((r_blk, k), row)] * 7,
        out_specs=[
            pl.BlockSpec((r_blk, k), row),
            pl.BlockSpec((r_blk, 1), row),
            pl.BlockSpec((r_blk, 1), row),
            pl.BlockSpec((r_blk, 3), row),
        ],
        out_shape=[
            jax.ShapeDtypeStruct((n, k), f32),
            jax.ShapeDtypeStruct((n, 1), f32),
            jax.ShapeDtypeStruct((n, 1), f32),
            jax.ShapeDtypeStruct((n, 3), f32),
        ],
    )(sigma2, tex0, tex1, tex2, sampled_depth, sampled_dists, sampled_idx)


def kernel(ray_start, ray_dir, sampled_depth, sampled_idx, sampled_dists, emb,
           W1, b1, W2, b2, Wsig, bsig, Wt1, bt1, Wt2, bt2):
    n, k = sampled_depth.shape
    nk = n * k
    hid = W2.shape[0]

    idx_flat = jnp.maximum(sampled_idx.reshape(nk), 0).astype(jnp.int32)
    v, de0 = emb.shape
    per = 256 // de0
    unit = _SC_SUBCORES * 56
    vq = -(-(v // per) // unit) * unit
    bf = jnp.bfloat16
    embp = jnp.pad(emb.astype(bf), ((0, vq * per - v), (0, 0))).reshape(vq, 256)
    lo = lax.bitcast_convert_type(embp[:, :128], jnp.uint16).astype(jnp.uint32)
    hi = lax.bitcast_convert_type(embp[:, 128:], jnp.uint16).astype(jnp.uint32)
    emb3 = lax.bitcast_convert_type(lo | (hi << 16), jnp.int32)
    g_flat = idx_flat // per
    q_flat = (idx_flat % per).reshape(nk, 1)

    rs_flat = jnp.broadcast_to(ray_start[:, None, :], (n, k, 3)).reshape(nk, 3)
    rd_flat = jnp.broadcast_to(ray_dir[:, None, :], (n, k, 3)).reshape(nk, 3)
    dep_flat = sampled_depth.reshape(nk, 1)

    w1b = W1[:39].astype(bf)
    w1c4 = jnp.concatenate([W1[39:]] * 4, axis=0).astype(bf)
    b1r, b2r = b1.reshape(1, -1), b2.reshape(1, -1)
    bsigr, bt1r, bt2r = bsig.reshape(1, 1), bt1.reshape(1, -1), bt2.reshape(1, -1)
    w2b, wsigb, wt2b = W2.astype(bf), Wsig.astype(bf), Wt2.astype(bf)
    wt1hb, wt1db = Wt1[:hid].astype(bf), Wt1[hid:].astype(bf)

    n_stripe = 8
    stripe = nk // n_stripe
    sigs, texs = [], []
    for s in range(n_stripe):
        sl = slice(s * stripe, (s + 1) * stripe)
        feat_s = _sc_gather(emb3, g_flat[sl])
        sg, tx = _mlp_call(
            rs_flat[sl], rd_flat[sl], dep_flat[sl], feat_s, q_flat[sl],
            w1b, w1c4, b1r, w2b, b2r, wsigb, bsigr,
            wt1hb, wt1db, bt1r, wt2b, bt2r, p_blk=4096)
        sigs.append(sg)
        texs.append(tx)
    sig = jnp.concatenate(sigs, axis=0)
    tex = jnp.concatenate(texs, axis=0)

    sigma2 = sig.reshape(n, k)
    tex3 = tex.reshape(n, k, 3)
    probs, depths, missed, colors = _render_call(
        sigma2, tex3[..., 0], tex3[..., 1], tex3[..., 2],
        sampled_depth, sampled_dists, sampled_idx, r_blk=512)
    return probs, depths.reshape(n), missed.reshape(n), colors


# speedup vs baseline: 4.4186x; 1.5391x over previous
import functools

import jax
import jax.numpy as jnp
import numpy as np
from jax import lax
from jax.experimental import pallas as pl
from jax.experimental.pallas import tpu as pltpu
from jax.experimental.pallas import tpu_sc as plsc

_SC_CORES = 2
_SC_SUBCORES = 16
_GATHER_CHUNK = 512
_L_PE = 6


def _sc_gather(emb3, g_flat):
    nk = g_flat.shape[0]
    vq, de = emb3.shape
    w = _GATHER_CHUNK
    mesh = plsc.VectorSubcoreMesh(core_axis_name="c", subcore_axis_name="s")

    rows_per_tile = vq // _SC_SUBCORES
    stage_rows = 56
    nw = _SC_CORES * _SC_SUBCORES
    b_per_w = nk // nw

    @functools.partial(
        pl.kernel,
        mesh=mesh,
        out_type=jax.ShapeDtypeStruct((nk, de), jnp.int32),
        scratch_types=[
            pltpu.VMEM_SHARED((vq, de), jnp.int32),
            pltpu.VMEM((stage_rows, de), jnp.int32),
            pltpu.VMEM((w,), jnp.int32),
            pltpu.VMEM((w, de), jnp.int32),
            pltpu.SemaphoreType.DMA,
        ],
    )
    def gather_kernel(emb_hbm, idx_hbm, out_hbm, sp_table, stage_v, idx_v, rows_v, sem):
        sbase = lax.axis_index("s") * rows_per_tile

        @pl.loop(0, rows_per_tile, step=stage_rows)
        def _(off):
            pltpu.sync_copy(emb_hbm.at[pl.ds(sbase + off, stage_rows)], stage_v)
            pltpu.sync_copy(stage_v, sp_table.at[pl.ds(sbase + off, stage_rows)])

        plsc.subcore_barrier()

        wid = lax.axis_index("s") * _SC_CORES + lax.axis_index("c")
        base = wid * b_per_w

        @pl.loop(0, b_per_w, step=w)
        def _(off):
            pltpu.sync_copy(idx_hbm.at[pl.ds(base + off, w)], idx_v)
            pltpu.async_copy(sp_table.at[idx_v], rows_v, sem).wait()
            pltpu.sync_copy(rows_v, out_hbm.at[pl.ds(base + off, w)])

    return gather_kernel(emb3, g_flat)


def _mlp_kernel(rs_ref, rd_ref, dep_ref, feat_ref, q_ref,
                w1_ref, w1c4_ref, b1_ref, w2_ref, b2_ref, wsig_ref, bsig_ref,
                wt1h_ref, wt1d_ref, bt1_ref, wt2_ref, bt2_ref,
                sig_ref, tex_ref):
    f32 = jnp.float32
    rd = rd_ref[...]
    pts = rs_ref[...] + rd * dep_ref[...]

    npe = 6 * _L_PE
    row3 = lax.broadcasted_iota(jnp.int32, (3, npe), 0)
    col3 = lax.broadcasted_iota(jnp.int32, (3, npe), 1)
    fr = jnp.exp2(((col3 % (2 * _L_PE)) % _L_PE).astype(f32)) * np.float32(np.pi)
    m = jnp.where(col3 // (2 * _L_PE) == row3, fr, 0.0)
    ang = jnp.dot(pts, m, precision=lax.Precision.HIGHEST, preferred_element_type=f32)
    colp = lax.broadcasted_iota(jnp.int32, (1, npe), 1)
    ph = jnp.where(colp % (2 * _L_PE) >= _L_PE, np.float32(0.5 * np.pi), np.float32(0.0))
    pe = jnp.sin(ang + ph)

    bf = jnp.bfloat16
    p = feat_ref.shape[0]
    fi = feat_ref[...]
    f_lo = lax.bitcast_convert_type(fi << 16, f32)
    f_hi = lax.bitcast_convert_type(fi & jnp.int32(-65536), f32)
    lane = lax.broadcasted_iota(jnp.int32, (p, 128), 1)
    q = q_ref[...]
    featm = jnp.where(lane // 32 == q, f_lo, 0.0) + jnp.where(lane // 32 == q - 4, f_hi, 0.0)
    acc = jnp.dot(pts.astype(bf), w1_ref[0:3, :], preferred_element_type=f32)
    acc += jnp.dot(pe.astype(bf), w1_ref[3:39, :], preferred_element_type=f32)
    acc += jnp.dot(featm.astype(bf), w1c4_ref[...], preferred_element_type=f32)
    h1 = jnp.maximum(acc + b1_ref[...], 0.0).astype(bf)
    h2 = jnp.maximum(jnp.dot(h1, w2_ref[...], preferred_element_type=f32) + b2_ref[...], 0.0).astype(bf)
    sig_ref[...] = jnp.dot(h2, wsig_ref[...], preferred_element_type=f32) + bsig_ref[...]
    t = jnp.dot(h2, wt1h_ref[...], preferred_element_type=f32)
    t += jnp.dot(rd.astype(bf), wt1d_ref[...], preferred_element_type=f32)
    t = jnp.maximum(t + bt1_ref[...], 0.0).astype(bf)
    tex_ref[...] = jax.nn.sigmoid(jnp.dot(t, wt2_ref[...], preferred_element_type=f32) + bt2_ref[...])


def _mlp_call(rs_flat, rd_flat, dep_flat, feat, q_flat,
              w1, w1c4, b1, w2, b2, wsig, bsig, wt1h, wt1d, bt1, wt2, bt2, p_blk):
    nk = rs_flat.shape[0]
    grid = nk // p_blk
    row = lambda i: (i, 0)
    rep = lambda i: (0, 0)
    f32 = jnp.float32
    return pl.pallas_call(
        _mlp_kernel,
        grid=(grid,),
        in_specs=[
            pl.BlockSpec((p_blk, 3), row),
            pl.BlockSpec((p_blk, 3), row),
            pl.BlockSpec((p_blk, 1), row),
            pl.BlockSpec((p_blk, feat.shape[1]), row),
            pl.BlockSpec((p_blk, 1), row),
            pl.BlockSpec(w1.shape, rep),
            pl.BlockSpec(w1c4.shape, rep),
            pl.BlockSpec(b1.shape, rep),
            pl.BlockSpec(w2.shape, rep),
            pl.BlockSpec(b2.shape, rep),
            pl.BlockSpec(wsig.shape, rep),
            pl.BlockSpec(bsig.shape, rep),
            pl.BlockSpec(wt1h.shape, rep),
            pl.BlockSpec(wt1d.shape, rep),
            pl.BlockSpec(bt1.shape, rep),
            pl.BlockSpec(wt2.shape, rep),
            pl.BlockSpec(bt2.shape, rep),
        ],
        out_specs=[
            pl.BlockSpec((p_blk, 1), row),
            pl.BlockSpec((p_blk, 3), row),
        ],
        out_shape=[
            jax.ShapeDtypeStruct((nk, 1), f32),
            jax.ShapeDtypeStruct((nk, 3), f32),
        ],
    )(rs_flat, rd_flat, dep_flat, feat, q_flat, w1, w1c4, b1, w2, b2, wsig, bsig,
      wt1h, wt1d, bt1, wt2, bt2)


def _render_kernel(sig_ref, t0_ref, t1_ref, t2_ref, dep_ref, dst_ref, idx_ref,
                   probs_ref, depth_ref, miss_ref, col_ref):
    k = sig_ref.shape[1]
    maskf = (idx_ref[...] != -1).astype(jnp.float32)
    fe = jnp.maximum(sig_ref[...], 0.0) * dst_ref[...] * maskf
    row = lax.broadcasted_iota(jnp.int32, (k, k), 0)
    col = lax.broadcasted_iota(jnp.int32, (k, k), 1)
    upper = (row < col).astype(jnp.float32)
    cum = jnp.dot(fe, upper, precision=lax.Precision.HIGHEST,
                  preferred_element_type=jnp.float32)
    probs = (1.0 - jnp.exp(-fe)) * jnp.exp(-cum)
    probs_ref[...] = probs
    depth_ref[...] = jnp.sum(dep_ref[...] * probs, axis=1, keepdims=True)
    miss_ref[...] = 1.0 - jnp.sum(probs, axis=1, keepdims=True)
    c0 = jnp.sum(t0_ref[...] * probs, axis=1, keepdims=True)
    c1 = jnp.sum(t1_ref[...] * probs, axis=1, keepdims=True)
    c2 = jnp.sum(t2_ref[...] * probs, axis=1, keepdims=True)
    col_ref[...] = jnp.concatenate([c0, c1, c2], axis=1)


def _render_call(sigma2, tex0, tex1, tex2, sampled_depth, sampled_dists, sampled_idx, r_blk):
    n, k = sigma2.shape
    grid = n // r_blk
    row = lambda i: (i, 0)
    f32 = jnp.float32
    return pl.pallas_call(
        _render_kernel,
        grid=(grid,),
        in_specs=[pl.BlockSpec((r_blk, k), row)] * 7,
        out_specs=[
            pl.BlockSpec((r_blk, k), row),
            pl.BlockSpec((r_blk, 1), row),
            pl.BlockSpec((r_blk, 1), row),
            pl.BlockSpec((r_blk, 3), row),
        ],
        out_shape=[
            jax.ShapeDtypeStruct((n, k), f32),
            jax.ShapeDtypeStruct((n, 1), f32),
            jax.ShapeDtypeStruct((n, 1), f32),
            jax.ShapeDtypeStruct((n, 3), f32),
        ],
    )(sigma2, tex0, tex1, tex2, sampled_depth, sampled_dists, sampled_idx)


def kernel(ray_start, ray_dir, sampled_depth, sampled_idx, sampled_dists, emb,
           W1, b1, W2, b2, Wsig, bsig, Wt1, bt1, Wt2, bt2):
    n, k = sampled_depth.shape
    nk = n * k
    hid = W2.shape[0]

    idx_flat = jnp.maximum(sampled_idx.reshape(nk), 0).astype(jnp.int32)
    v, de0 = emb.shape
    per = 256 // de0
    unit = _SC_SUBCORES * 56
    vq = -(-(v // per) // unit) * unit
    bf = jnp.bfloat16
    embp = jnp.pad(emb.astype(bf), ((0, vq * per - v), (0, 0))).reshape(vq, 256)
    lo = lax.bitcast_convert_type(embp[:, :128], jnp.uint16).astype(jnp.uint32)
    hi = lax.bitcast_convert_type(embp[:, 128:], jnp.uint16).astype(jnp.uint32)
    emb3 = lax.bitcast_convert_type(lo | (hi << 16), jnp.int32)
    g_flat = idx_flat // per
    q_flat = (idx_flat % per).reshape(nk, 1)

    rs_flat = jnp.broadcast_to(ray_start[:, None, :], (n, k, 3)).reshape(nk, 3)
    rd_flat = jnp.broadcast_to(ray_dir[:, None, :], (n, k, 3)).reshape(nk, 3)
    dep_flat = sampled_depth.reshape(nk, 1)

    w1b = W1[:39].astype(bf)
    w1c4 = jnp.concatenate([W1[39:]] * 4, axis=0).astype(bf)
    b1r, b2r = b1.reshape(1, -1), b2.reshape(1, -1)
    bsigr, bt1r, bt2r = bsig.reshape(1, 1), bt1.reshape(1, -1), bt2.reshape(1, -1)
    w2b, wsigb, wt2b = W2.astype(bf), Wsig.astype(bf), Wt2.astype(bf)
    wt1hb, wt1db = Wt1[:hid].astype(bf), Wt1[hid:].astype(bf)

    n_stripe = 8
    stripe = nk // n_stripe
    sigs, texs = [], []
    for s in range(n_stripe):
        sl = slice(s * stripe, (s + 1) * stripe)
        feat_s = _sc_gather(emb3, g_flat[sl])
        sg, tx = _mlp_call(
            rs_flat[sl], rd_flat[sl], dep_flat[sl], feat_s, q_flat[sl],
            w1b, w1c4, b1r, w2b, b2r, wsigb, bsigr,
            wt1hb, wt1db, bt1r, wt2b, bt2r, p_blk=4096)
        sigs.append(sg)
        texs.append(tx)
    sig = jnp.concatenate(sigs, axis=0)
    tex = jnp.concatenate(texs, axis=0)

    sigma2 = sig.reshape(n, k)
    tex3 = tex.reshape(n, k, 3)
    probs, depths, missed, colors = _render_call(
        sigma2, tex3[..., 0], tex3[..., 1], tex3[..., 2],
        sampled_depth, sampled_dists, sampled_idx, r_blk=512)
    return probs, depths.reshape(n), missed.reshape(n), colors


# speedup vs baseline: 6.1691x; 1.3962x over previous
import functools

import jax
import jax.numpy as jnp
import numpy as np
from jax import lax
from jax.experimental import pallas as pl
from jax.experimental.pallas import tpu as pltpu
from jax.experimental.pallas import tpu_sc as plsc

_SC_CORES = 2
_SC_SUBCORES = 16
_GATHER_CHUNK = 512
_L_PE = 6


def _sc_gather(emb3, g_flat):
    nk = g_flat.shape[0]
    vq, de = emb3.shape
    w = _GATHER_CHUNK
    mesh = plsc.VectorSubcoreMesh(core_axis_name="c", subcore_axis_name="s")

    rows_per_tile = vq // _SC_SUBCORES
    stage_rows = 56
    nw = _SC_CORES * _SC_SUBCORES
    b_per_w = nk // nw

    @functools.partial(
        pl.kernel,
        mesh=mesh,
        out_type=jax.ShapeDtypeStruct((nk, de), jnp.int32),
        scratch_types=[
            pltpu.VMEM_SHARED((vq, de), jnp.int32),
            pltpu.VMEM((stage_rows, de), jnp.int32),
            pltpu.VMEM((w,), jnp.int32),
            pltpu.VMEM((w, de), jnp.int32),
            pltpu.SemaphoreType.DMA,
        ],
    )
    def gather_kernel(emb_hbm, idx_hbm, out_hbm, sp_table, stage_v, idx_v, rows_v, sem):
        sbase = lax.axis_index("s") * rows_per_tile

        @pl.loop(0, rows_per_tile, step=stage_rows)
        def _(off):
            pltpu.sync_copy(emb_hbm.at[pl.ds(sbase + off, stage_rows)], stage_v)
            pltpu.sync_copy(stage_v, sp_table.at[pl.ds(sbase + off, stage_rows)])

        plsc.subcore_barrier()

        wid = lax.axis_index("s") * _SC_CORES + lax.axis_index("c")
        base = wid * b_per_w

        @pl.loop(0, b_per_w, step=w)
        def _(off):
            pltpu.sync_copy(idx_hbm.at[pl.ds(base + off, w)], idx_v)
            pltpu.async_copy(sp_table.at[idx_v], rows_v, sem).wait()
            pltpu.sync_copy(rows_v, out_hbm.at[pl.ds(base + off, w)])

    return gather_kernel(emb3, g_flat)


def _mlp_kernel(geom_ref, feat_ref,
                w1_ref, w1c4_ref, b1_ref, w2_ref, b2_ref, wsig_ref, bsig_ref,
                wt1h_ref, wt1d_ref, bt1_ref, wt2_ref, bt2_ref,
                out_ref):
    f32 = jnp.float32
    geom = geom_ref[...]
    rd = geom[:, 3:6]
    pts = geom[:, 0:3] + rd * geom[:, 6:7]

    npe = 6 * _L_PE
    row3 = lax.broadcasted_iota(jnp.int32, (3, npe), 0)
    col3 = lax.broadcasted_iota(jnp.int32, (3, npe), 1)
    fr = jnp.exp2(((col3 % (2 * _L_PE)) % _L_PE).astype(f32)) * np.float32(np.pi)
    m = jnp.where(col3 // (2 * _L_PE) == row3, fr, 0.0)
    ang = jnp.dot(pts, m, precision=lax.Precision.HIGHEST, preferred_element_type=f32)
    colp = lax.broadcasted_iota(jnp.int32, (1, npe), 1)
    ph = jnp.where(colp % (2 * _L_PE) >= _L_PE, np.float32(0.5 * np.pi), np.float32(0.0))
    pe = jnp.sin(ang + ph)

    bf = jnp.bfloat16
    p = feat_ref.shape[0]
    fi = feat_ref[...]
    f_lo = lax.bitcast_convert_type(fi << 16, f32)
    f_hi = lax.bitcast_convert_type(fi & jnp.int32(-65536), f32)
    lane = lax.broadcasted_iota(jnp.int32, (p, 128), 1)
    q = geom[:, 7:8].astype(jnp.int32)
    featm = jnp.where(lane // 32 == q, f_lo, 0.0) + jnp.where(lane // 32 == q - 4, f_hi, 0.0)
    acc = jnp.dot(pts.astype(bf), w1_ref[0:3, :], preferred_element_type=f32)
    acc += jnp.dot(pe.astype(bf), w1_ref[3:39, :], preferred_element_type=f32)
    acc += jnp.dot(featm.astype(bf), w1c4_ref[...], preferred_element_type=f32)
    h1 = jnp.maximum(acc + b1_ref[...], 0.0).astype(bf)
    h2 = jnp.maximum(jnp.dot(h1, w2_ref[...], preferred_element_type=f32) + b2_ref[...], 0.0).astype(bf)
    out_ref[:, 0:1] = jnp.dot(h2, wsig_ref[...], preferred_element_type=f32) + bsig_ref[...]
    t = jnp.dot(h2, wt1h_ref[...], preferred_element_type=f32)
    t += jnp.dot(rd.astype(bf), wt1d_ref[...], preferred_element_type=f32)
    t = jnp.maximum(t + bt1_ref[...], 0.0).astype(bf)
    out_ref[:, 1:4] = jax.nn.sigmoid(jnp.dot(t, wt2_ref[...], preferred_element_type=f32) + bt2_ref[...])


def _mlp_call(geom, feat,
              w1, w1c4, b1, w2, b2, wsig, bsig, wt1h, wt1d, bt1, wt2, bt2, p_blk):
    nk = geom.shape[0]
    grid = nk // p_blk
    row = lambda i: (i, 0)
    rep = lambda i: (0, 0)
    f32 = jnp.float32
    return pl.pallas_call(
        _mlp_kernel,
        grid=(grid,),
        in_specs=[
            pl.BlockSpec((p_blk, geom.shape[1]), row),
            pl.BlockSpec((p_blk, feat.shape[1]), row),
            pl.BlockSpec(w1.shape, rep),
            pl.BlockSpec(w1c4.shape, rep),
            pl.BlockSpec(b1.shape, rep),
            pl.BlockSpec(w2.shape, rep),
            pl.BlockSpec(b2.shape, rep),
            pl.BlockSpec(wsig.shape, rep),
            pl.BlockSpec(bsig.shape, rep),
            pl.BlockSpec(wt1h.shape, rep),
            pl.BlockSpec(wt1d.shape, rep),
            pl.BlockSpec(bt1.shape, rep),
            pl.BlockSpec(wt2.shape, rep),
            pl.BlockSpec(bt2.shape, rep),
        ],
        out_specs=[pl.BlockSpec((p_blk, 4), row)],
        out_shape=[jax.ShapeDtypeStruct((nk, 4), f32)],
    )(geom, feat, w1, w1c4, b1, w2, b2, wsig, bsig,
      wt1h, wt1d, bt1, wt2, bt2)[0]


def _render_kernel(sig_ref, t0_ref, t1_ref, t2_ref, dep_ref, dst_ref, idx_ref,
                   probs_ref, depth_ref, miss_ref, col_ref):
    k = sig_ref.shape[1]
    maskf = (idx_ref[...] != -1).astype(jnp.float32)
    fe = jnp.maximum(sig_ref[...], 0.0) * dst_ref[...] * maskf
    row = lax.broadcasted_iota(jnp.int32, (k, k), 0)
    col = lax.broadcasted_iota(jnp.int32, (k, k), 1)
    upper = (row < col).astype(jnp.float32)
    cum = jnp.dot(fe, upper, precision=lax.Precision.HIGHEST,
                  preferred_element_type=jnp.float32)
    probs = (1.0 - jnp.exp(-fe)) * jnp.exp(-cum)
    probs_ref[...] = probs
    depth_ref[...] = jnp.sum(dep_ref[...] * probs, axis=1, keepdims=True)
    miss_ref[...] = 1.0 - jnp.sum(probs, axis=1, keepdims=True)
    c0 = jnp.sum(t0_ref[...] * probs, axis=1, keepdims=True)
    c1 = jnp.sum(t1_ref[...] * probs, axis=1, keepdims=True)
    c2 = jnp.sum(t2_ref[...] * probs, axis=1, keepdims=True)
    col_ref[...] = jnp.concatenate([c0, c1, c2], axis=1)


def _render_call(sigma2, tex0, tex1, tex2, sampled_depth, sampled_dists, sampled_idx, r_blk):
    n, k = sigma2.shape
    grid = n // r_blk
    row = lambda i: (i, 0)
    f32 = jnp.float32
    return pl.pallas_call(
        _render_kernel,
        grid=(grid,),
        in_specs=[pl.BlockSpec((r_blk, k), row)] * 7,
        out_specs=[
            pl.BlockSpec((r_blk, k), row),
            pl.BlockSpec((r_blk, 1), row),
            pl.BlockSpec((r_blk, 1), row),
            pl.BlockSpec((r_blk, 3), row),
        ],
        out_shape=[
            jax.ShapeDtypeStruct((n, k), f32),
            jax.ShapeDtypeStruct((n, 1), f32),
            jax.ShapeDtypeStruct((n, 1), f32),
            jax.ShapeDtypeStruct((n, 3), f32),
        ],
    )(sigma2, tex0, tex1, tex2, sampled_depth, sampled_dists, sampled_idx)


def kernel(ray_start, ray_dir, sampled_depth, sampled_idx, sampled_dists, emb,
           W1, b1, W2, b2, Wsig, bsig, Wt1, bt1, Wt2, bt2):
    n, k = sampled_depth.shape
    nk = n * k
    hid = W2.shape[0]

    idx_flat = jnp.maximum(sampled_idx.reshape(nk), 0).astype(jnp.int32)
    v, de0 = emb.shape
    per = 256 // de0
    unit = _SC_SUBCORES * 56
    vq = -(-(v // per) // unit) * unit
    bf = jnp.bfloat16
    embp = jnp.pad(emb.astype(bf), ((0, vq * per - v), (0, 0))).reshape(vq, 256)
    lo = lax.bitcast_convert_type(embp[:, :128], jnp.uint16).astype(jnp.uint32)
    hi = lax.bitcast_convert_type(embp[:, 128:], jnp.uint16).astype(jnp.uint32)
    emb3 = lax.bitcast_convert_type(lo | (hi << 16), jnp.int32)
    g_flat = idx_flat // per
    q_flat = (idx_flat % per).astype(jnp.float32).reshape(nk, 1)

    rs_flat = jnp.broadcast_to(ray_start[:, None, :], (n, k, 3)).reshape(nk, 3)
    rd_flat = jnp.broadcast_to(ray_dir[:, None, :], (n, k, 3)).reshape(nk, 3)
    dep_flat = sampled_depth.reshape(nk, 1)
    geom = jnp.concatenate([rs_flat, rd_flat, dep_flat, q_flat], axis=1)

    w1b = W1[:39].astype(bf)
    w1c4 = jnp.concatenate([W1[39:]] * 4, axis=0).astype(bf)
    b1r, b2r = b1.reshape(1, -1), b2.reshape(1, -1)
    bsigr, bt1r, bt2r = bsig.reshape(1, 1), bt1.reshape(1, -1), bt2.reshape(1, -1)
    w2b, wsigb, wt2b = W2.astype(bf), Wsig.astype(bf), Wt2.astype(bf)
    wt1hb, wt1db = Wt1[:hid].astype(bf), Wt1[hid:].astype(bf)

    n_stripe = 8
    stripe = nk // n_stripe
    outs = []
    for s in range(n_stripe):
        sl = slice(s * stripe, (s + 1) * stripe)
        feat_s = _sc_gather(emb3, g_flat[sl])
        outs.append(_mlp_call(
            geom[sl], feat_s,
            w1b, w1c4, b1r, w2b, b2r, wsigb, bsigr,
            wt1hb, wt1db, bt1r, wt2b, bt2r, p_blk=4096))
    st = jnp.concatenate(outs, axis=0).reshape(n, k, 4)

    probs, depths, missed, colors = _render_call(
        st[..., 0], st[..., 1], st[..., 2], st[..., 3],
        sampled_depth, sampled_dists, sampled_idx, r_blk=512)
    return probs, depths.reshape(n), missed.reshape(n), colors


# speedup vs baseline: 8.5641x; 1.3882x over previous
import functools

import jax
import jax.numpy as jnp
import numpy as np
from jax import lax
from jax.experimental import pallas as pl
from jax.experimental.pallas import tpu as pltpu
from jax.experimental.pallas import tpu_sc as plsc

_SC_CORES = 2
_SC_SUBCORES = 16
_GATHER_CHUNK = 512
_L_PE = 6


def _sc_gather(emb3, g_flat):
    nk = g_flat.shape[0]
    vq, de = emb3.shape
    w = _GATHER_CHUNK
    mesh = plsc.VectorSubcoreMesh(core_axis_name="c", subcore_axis_name="s")

    rows_per_tile = vq // _SC_SUBCORES
    stage_rows = 56
    nw = _SC_CORES * _SC_SUBCORES
    b_per_w = nk // nw

    @functools.partial(
        pl.kernel,
        mesh=mesh,
        out_type=jax.ShapeDtypeStruct((nk, de), jnp.int32),
        scratch_types=[
            pltpu.VMEM_SHARED((vq, de), jnp.int32),
            pltpu.VMEM((stage_rows, de), jnp.int32),
            pltpu.VMEM((w,), jnp.int32),
            pltpu.VMEM((w, de), jnp.int32),
            pltpu.SemaphoreType.DMA,
        ],
    )
    def gather_kernel(emb_hbm, idx_hbm, out_hbm, sp_table, stage_v, idx_v, rows_v, sem):
        sbase = lax.axis_index("s") * rows_per_tile

        @pl.loop(0, rows_per_tile, step=stage_rows)
        def _(off):
            pltpu.sync_copy(emb_hbm.at[pl.ds(sbase + off, stage_rows)], stage_v)
            pltpu.sync_copy(stage_v, sp_table.at[pl.ds(sbase + off, stage_rows)])

        plsc.subcore_barrier()

        wid = lax.axis_index("s") * _SC_CORES + lax.axis_index("c")
        base = wid * b_per_w

        @pl.loop(0, b_per_w, step=w)
        def _(off):
            pltpu.sync_copy(idx_hbm.at[pl.ds(base + off, w)], idx_v)
            pltpu.async_copy(sp_table.at[idx_v], rows_v, sem).wait()
            pltpu.sync_copy(rows_v, out_hbm.at[pl.ds(base + off, w)])

    return gather_kernel(emb3, g_flat)


def _mlp_kernel(geom_ref, feat_ref,
                w1_ref, w1c4_ref, b1_ref, w2_ref, b2_ref, wsig_ref, bsig_ref,
                wt1h_ref, wt1d_ref, bt1_ref, wt2_ref, bt2_ref,
                out_ref):
    f32 = jnp.float32
    geom = geom_ref[...]
    rd = geom[:, 3:6]
    pts = geom[:, 0:3] + rd * geom[:, 6:7]

    npe = 6 * _L_PE
    row3 = lax.broadcasted_iota(jnp.int32, (3, npe), 0)
    col3 = lax.broadcasted_iota(jnp.int32, (3, npe), 1)
    fr = jnp.exp2(((col3 % (2 * _L_PE)) % _L_PE).astype(f32))
    m = jnp.where(col3 // (2 * _L_PE) == row3, fr, 0.0)
    vang = jnp.dot(pts, m, precision=lax.Precision.HIGHEST, preferred_element_type=f32)
    colp = lax.broadcasted_iota(jnp.int32, (1, npe), 1)
    ph = jnp.where(colp % (2 * _L_PE) >= _L_PE, np.float32(0.5), np.float32(0.0))
    v = vang + ph
    r = v - 2.0 * jnp.round(0.5 * v)
    r2 = r * r
    pe = r * (3.1415442525 + r2 * (-5.1666563063 + r2 * (2.5437544412
              + r2 * (-0.5834079913 + r2 * 0.0647828326))))

    bf = jnp.bfloat16
    p = feat_ref.shape[0]
    fi = feat_ref[...]
    f_lo = lax.bitcast_convert_type(fi << 16, f32)
    f_hi = lax.bitcast_convert_type(fi & jnp.int32(-65536), f32)
    lane = lax.broadcasted_iota(jnp.int32, (p, 128), 1)
    q = geom[:, 7:8].astype(jnp.int32)
    featm = jnp.where(lane // 32 == q, f_lo, 0.0) + jnp.where(lane // 32 == q - 4, f_hi, 0.0)
    acc = jnp.dot(pts.astype(bf), w1_ref[0:3, :], preferred_element_type=f32)
    acc += jnp.dot(pe.astype(bf), w1_ref[3:39, :], preferred_element_type=f32)
    acc += jnp.dot(featm.astype(bf), w1c4_ref[...], preferred_element_type=f32)
    h1 = jnp.maximum(acc + b1_ref[...], 0.0).astype(bf)
    h2 = jnp.maximum(jnp.dot(h1, w2_ref[...], preferred_element_type=f32) + b2_ref[...], 0.0).astype(bf)
    out_ref[:, 0:1] = jnp.dot(h2, wsig_ref[...], preferred_element_type=f32) + bsig_ref[...]
    t = jnp.dot(h2, wt1h_ref[...], preferred_element_type=f32)
    t += jnp.dot(rd.astype(bf), wt1d_ref[...], preferred_element_type=f32)
    t = jnp.maximum(t + bt1_ref[...], 0.0).astype(bf)
    out_ref[:, 1:4] = jax.nn.sigmoid(jnp.dot(t, wt2_ref[...], preferred_element_type=f32) + bt2_ref[...])


def _mlp_call(geom, feat,
              w1, w1c4, b1, w2, b2, wsig, bsig, wt1h, wt1d, bt1, wt2, bt2, p_blk):
    nk = geom.shape[0]
    grid = nk // p_blk
    row = lambda i: (i, 0)
    rep = lambda i: (0, 0)
    f32 = jnp.float32
    return pl.pallas_call(
        _mlp_kernel,
        grid=(grid,),
        in_specs=[
            pl.BlockSpec((p_blk, geom.shape[1]), row),
            pl.BlockSpec((p_blk, feat.shape[1]), row),
            pl.BlockSpec(w1.shape, rep),
            pl.BlockSpec(w1c4.shape, rep),
            pl.BlockSpec(b1.shape, rep),
            pl.BlockSpec(w2.shape, rep),
            pl.BlockSpec(b2.shape, rep),
            pl.BlockSpec(wsig.shape, rep),
            pl.BlockSpec(bsig.shape, rep),
            pl.BlockSpec(wt1h.shape, rep),
            pl.BlockSpec(wt1d.shape, rep),
            pl.BlockSpec(bt1.shape, rep),
            pl.BlockSpec(wt2.shape, rep),
            pl.BlockSpec(bt2.shape, rep),
        ],
        out_specs=[pl.BlockSpec((p_blk, 4), row)],
        out_shape=[jax.ShapeDtypeStruct((nk, 4), f32)],
    )(geom, feat, w1, w1c4, b1, w2, b2, wsig, bsig,
      wt1h, wt1d, bt1, wt2, bt2)[0]


def _render_kernel(sig_ref, t0_ref, t1_ref, t2_ref, dep_ref, dst_ref, idx_ref,
                   probs_ref, depth_ref, miss_ref, col_ref):
    k = sig_ref.shape[1]
    maskf = (idx_ref[...] != -1).astype(jnp.float32)
    fe = jnp.maximum(sig_ref[...], 0.0) * dst_ref[...] * maskf
    row = lax.broadcasted_iota(jnp.int32, (k, k), 0)
    col = lax.broadcasted_iota(jnp.int32, (k, k), 1)
    upper = (row < col).astype(jnp.float32)
    cum = jnp.dot(fe, upper, precision=lax.Precision.HIGHEST,
                  preferred_element_type=jnp.float32)
    probs = (1.0 - jnp.exp(-fe)) * jnp.exp(-cum)
    probs_ref[...] = probs
    depth_ref[...] = jnp.sum(dep_ref[...] * probs, axis=1, keepdims=True)
    miss_ref[...] = 1.0 - jnp.sum(probs, axis=1, keepdims=True)
    c0 = jnp.sum(t0_ref[...] * probs, axis=1, keepdims=True)
    c1 = jnp.sum(t1_ref[...] * probs, axis=1, keepdims=True)
    c2 = jnp.sum(t2_ref[...] * probs, axis=1, keepdims=True)
    col_ref[...] = jnp.concatenate([c0, c1, c2], axis=1)


def _render_call(sigma2, tex0, tex1, tex2, sampled_depth, sampled_dists, sampled_idx, r_blk):
    n, k = sigma2.shape
    grid = n // r_blk
    row = lambda i: (i, 0)
    f32 = jnp.float32
    return pl.pallas_call(
        _render_kernel,
        grid=(grid,),
        in_specs=[pl.BlockSpec((r_blk, k), row)] * 7,
        out_specs=[
            pl.BlockSpec((r_blk, k), row),
            pl.BlockSpec((r_blk, 1), row),
            pl.BlockSpec((r_blk, 1), row),
            pl.BlockSpec((r_blk, 3), row),
        ],
        out_shape=[
            jax.ShapeDtypeStruct((n, k), f32),
            jax.ShapeDtypeStruct((n, 1), f32),
            jax.ShapeDtypeStruct((n, 1), f32),
            jax.ShapeDtypeStruct((n, 3), f32),
        ],
    )(sigma2, tex0, tex1, tex2, sampled_depth, sampled_dists, sampled_idx)


def kernel(ray_start, ray_dir, sampled_depth, sampled_idx, sampled_dists, emb,
           W1, b1, W2, b2, Wsig, bsig, Wt1, bt1, Wt2, bt2):
    n, k = sampled_depth.shape
    nk = n * k
    hid = W2.shape[0]

    idx_flat = jnp.maximum(sampled_idx.reshape(nk), 0).astype(jnp.int32)
    v, de0 = emb.shape
    per = 256 // de0
    unit = _SC_SUBCORES * 56
    vq = -(-(v // per) // unit) * unit
    bf = jnp.bfloat16
    embp = jnp.pad(emb.astype(bf), ((0, vq * per - v), (0, 0))).reshape(vq, 256)
    lo = lax.bitcast_convert_type(embp[:, :128], jnp.uint16).astype(jnp.uint32)
    hi = lax.bitcast_convert_type(embp[:, 128:], jnp.uint16).astype(jnp.uint32)
    emb3 = lax.bitcast_convert_type(lo | (hi << 16), jnp.int32)
    g_flat = idx_flat // per
    q_flat = (idx_flat % per).astype(jnp.float32).reshape(nk, 1)

    rs_flat = jnp.broadcast_to(ray_start[:, None, :], (n, k, 3)).reshape(nk, 3)
    rd_flat = jnp.broadcast_to(ray_dir[:, None, :], (n, k, 3)).reshape(nk, 3)
    dep_flat = sampled_depth.reshape(nk, 1)
    geom = jnp.concatenate([rs_flat, rd_flat, dep_flat, q_flat], axis=1)

    w1b = W1[:39].astype(bf)
    w1c4 = jnp.concatenate([W1[39:]] * 4, axis=0).astype(bf)
    b1r, b2r = b1.reshape(1, -1), b2.reshape(1, -1)
    bsigr, bt1r, bt2r = bsig.reshape(1, 1), bt1.reshape(1, -1), bt2.reshape(1, -1)
    w2b, wsigb, wt2b = W2.astype(bf), Wsig.astype(bf), Wt2.astype(bf)
    wt1hb, wt1db = Wt1[:hid].astype(bf), Wt1[hid:].astype(bf)

    n_stripe = 8
    stripe = nk // n_stripe
    outs = []
    for s in range(n_stripe):
        sl = slice(s * stripe, (s + 1) * stripe)
        feat_s = _sc_gather(emb3, g_flat[sl])
        outs.append(_mlp_call(
            geom[sl], feat_s,
            w1b, w1c4, b1r, w2b, b2r, wsigb, bsigr,
            wt1hb, wt1db, bt1r, wt2b, bt2r, p_blk=4096))
    st = jnp.concatenate(outs, axis=0).reshape(n, k, 4)

    probs, depths, missed, colors = _render_call(
        st[..., 0], st[..., 1], st[..., 2], st[..., 3],
        sampled_depth, sampled_dists, sampled_idx, r_blk=512)
    return probs, depths.reshape(n), missed.reshape(n), colors


# speedup vs baseline: 10.0063x; 1.1684x over previous
import functools

import jax
import jax.numpy as jnp
import numpy as np
from jax import lax
from jax.experimental import pallas as pl
from jax.experimental.pallas import tpu as pltpu
from jax.experimental.pallas import tpu_sc as plsc

_SC_CORES = 2
_SC_SUBCORES = 16
_GATHER_CHUNK = 512
_L_PE = 6


def _sc_gather(emb3, g_flat):
    nk = g_flat.shape[0]
    vq, de = emb3.shape
    w = _GATHER_CHUNK
    mesh = plsc.VectorSubcoreMesh(core_axis_name="c", subcore_axis_name="s")

    rows_per_tile = vq // _SC_SUBCORES
    stage_rows = 56
    nw = _SC_CORES * _SC_SUBCORES
    b_per_w = nk // nw

    @functools.partial(
        pl.kernel,
        mesh=mesh,
        out_type=jax.ShapeDtypeStruct((nk, de), jnp.int32),
        scratch_types=[
            pltpu.VMEM_SHARED((vq, de), jnp.int32),
            pltpu.VMEM((stage_rows, de), jnp.int32),
            pltpu.VMEM((w,), jnp.int32),
            pltpu.VMEM((w, de), jnp.int32),
            pltpu.SemaphoreType.DMA,
        ],
    )
    def gather_kernel(emb_hbm, idx_hbm, out_hbm, sp_table, stage_v, idx_v, rows_v, sem):
        sbase = lax.axis_index("s") * rows_per_tile

        @pl.loop(0, rows_per_tile, step=stage_rows)
        def _(off):
            pltpu.sync_copy(emb_hbm.at[pl.ds(sbase + off, stage_rows)], stage_v)
            pltpu.sync_copy(stage_v, sp_table.at[pl.ds(sbase + off, stage_rows)])

        plsc.subcore_barrier()

        wid = lax.axis_index("s") * _SC_CORES + lax.axis_index("c")
        base = wid * b_per_w

        @pl.loop(0, b_per_w, step=w)
        def _(off):
            pltpu.sync_copy(idx_hbm.at[pl.ds(base + off, w)], idx_v)
            pltpu.async_copy(sp_table.at[idx_v], rows_v, sem).wait()
            pltpu.sync_copy(rows_v, out_hbm.at[pl.ds(base + off, w)])

    return gather_kernel(emb3, g_flat)


def _mlp_kernel(geom_ref, feat_ref,
                w1_ref, w1c4_ref, b1_ref, w2_ref, b2_ref, wsig_ref, bsig_ref,
                wt1h_ref, wt1d_ref, bt1_ref, wt2_ref, bt2_ref,
                out_ref):
    f32 = jnp.float32
    geom = geom_ref[...]
    rd = geom[:, 3:6]
    pts = geom[:, 0:3] + rd * geom[:, 6:7]

    npe = 6 * _L_PE
    row3 = lax.broadcasted_iota(jnp.int32, (3, npe), 0)
    col3 = lax.broadcasted_iota(jnp.int32, (3, npe), 1)
    fr = jnp.exp2(((col3 % (2 * _L_PE)) % _L_PE).astype(f32))
    m = jnp.where(col3 // (2 * _L_PE) == row3, fr, 0.0)
    bfs = jnp.bfloat16
    mb = m.astype(bfs)
    p1 = pts.astype(bfs)
    rem = pts - p1.astype(f32)
    p2 = rem.astype(bfs)
    p3 = (rem - p2.astype(f32)).astype(bfs)
    vang = (jnp.dot(p1, mb, preferred_element_type=f32)
            + jnp.dot(p2, mb, preferred_element_type=f32)
            + jnp.dot(p3, mb, preferred_element_type=f32))
    colp = lax.broadcasted_iota(jnp.int32, (1, npe), 1)
    ph = jnp.where(colp % (2 * _L_PE) >= _L_PE, np.float32(0.5), np.float32(0.0))
    v = vang + ph
    r = v - 2.0 * jnp.round(0.5 * v)
    r2 = r * r
    pe = r * (3.1415442525 + r2 * (-5.1666563063 + r2 * (2.5437544412
              + r2 * (-0.5834079913 + r2 * 0.0647828326))))

    bf = jnp.bfloat16
    p = feat_ref.shape[0]
    fi = feat_ref[...]
    f_lo = lax.bitcast_convert_type(fi << 16, f32)
    f_hi = lax.bitcast_convert_type(fi & jnp.int32(-65536), f32)
    lane = lax.broadcasted_iota(jnp.int32, (p, 128), 1)
    q = geom[:, 7:8].astype(jnp.int32)
    featm = jnp.where(lane // 32 == q, f_lo, 0.0) + jnp.where(lane // 32 == q - 4, f_hi, 0.0)
    acc = jnp.dot(pts.astype(bf), w1_ref[0:3, :], preferred_element_type=f32)
    acc += jnp.dot(pe.astype(bf), w1_ref[3:39, :], preferred_element_type=f32)
    acc += jnp.dot(featm.astype(bf), w1c4_ref[...], preferred_element_type=f32)
    h1 = jnp.maximum(acc + b1_ref[...], 0.0).astype(bf)
    h2 = jnp.maximum(jnp.dot(h1, w2_ref[...], preferred_element_type=f32) + b2_ref[...], 0.0).astype(bf)
    out_ref[:, 0:1] = jnp.dot(h2, wsig_ref[...], preferred_element_type=f32) + bsig_ref[...]
    t = jnp.dot(h2, wt1h_ref[...], preferred_element_type=f32)
    t += jnp.dot(rd.astype(bf), wt1d_ref[...], preferred_element_type=f32)
    t = jnp.maximum(t + bt1_ref[...], 0.0).astype(bf)
    out_ref[:, 1:4] = jax.nn.sigmoid(jnp.dot(t, wt2_ref[...], preferred_element_type=f32) + bt2_ref[...])


def _mlp_call(geom, feat,
              w1, w1c4, b1, w2, b2, wsig, bsig, wt1h, wt1d, bt1, wt2, bt2, p_blk):
    nk = geom.shape[0]
    grid = nk // p_blk
    row = lambda i: (i, 0)
    rep = lambda i: (0, 0)
    f32 = jnp.float32
    return pl.pallas_call(
        _mlp_kernel,
        grid=(grid,),
        in_specs=[
            pl.BlockSpec((p_blk, geom.shape[1]), row),
            pl.BlockSpec((p_blk, feat.shape[1]), row),
            pl.BlockSpec(w1.shape, rep),
            pl.BlockSpec(w1c4.shape, rep),
            pl.BlockSpec(b1.shape, rep),
            pl.BlockSpec(w2.shape, rep),
            pl.BlockSpec(b2.shape, rep),
            pl.BlockSpec(wsig.shape, rep),
            pl.BlockSpec(bsig.shape, rep),
            pl.BlockSpec(wt1h.shape, rep),
            pl.BlockSpec(wt1d.shape, rep),
            pl.BlockSpec(bt1.shape, rep),
            pl.BlockSpec(wt2.shape, rep),
            pl.BlockSpec(bt2.shape, rep),
        ],
        out_specs=[pl.BlockSpec((p_blk, 4), row)],
        out_shape=[jax.ShapeDtypeStruct((nk, 4), f32)],
    )(geom, feat, w1, w1c4, b1, w2, b2, wsig, bsig,
      wt1h, wt1d, bt1, wt2, bt2)[0]


def _render_kernel(sig_ref, t0_ref, t1_ref, t2_ref, dep_ref, dst_ref, idx_ref,
                   probs_ref, depth_ref, miss_ref, col_ref):
    k = sig_ref.shape[1]
    maskf = (idx_ref[...] != -1).astype(jnp.float32)
    fe = jnp.maximum(sig_ref[...], 0.0) * dst_ref[...] * maskf
    row = lax.broadcasted_iota(jnp.int32, (k, k), 0)
    col = lax.broadcasted_iota(jnp.int32, (k, k), 1)
    upper = (row < col).astype(jnp.float32)
    cum = jnp.dot(fe, upper, precision=lax.Precision.HIGHEST,
                  preferred_element_type=jnp.float32)
    probs = (1.0 - jnp.exp(-fe)) * jnp.exp(-cum)
    probs_ref[...] = probs
    depth_ref[...] = jnp.sum(dep_ref[...] * probs, axis=1, keepdims=True)
    miss_ref[...] = 1.0 - jnp.sum(probs, axis=1, keepdims=True)
    c0 = jnp.sum(t0_ref[...] * probs, axis=1, keepdims=True)
    c1 = jnp.sum(t1_ref[...] * probs, axis=1, keepdims=True)
    c2 = jnp.sum(t2_ref[...] * probs, axis=1, keepdims=True)
    col_ref[...] = jnp.concatenate([c0, c1, c2], axis=1)


def _render_call(sigma2, tex0, tex1, tex2, sampled_depth, sampled_dists, sampled_idx, r_blk):
    n, k = sigma2.shape
    grid = n // r_blk
    row = lambda i: (i, 0)
    f32 = jnp.float32
    return pl.pallas_call(
        _render_kernel,
        grid=(grid,),
        in_specs=[pl.BlockSpec((r_blk, k), row)] * 7,
        out_specs=[
            pl.BlockSpec((r_blk, k), row),
            pl.BlockSpec((r_blk, 1), row),
            pl.BlockSpec((r_blk, 1), row),
            pl.BlockSpec((r_blk, 3), row),
        ],
        out_shape=[
            jax.ShapeDtypeStruct((n, k), f32),
            jax.ShapeDtypeStruct((n, 1), f32),
            jax.ShapeDtypeStruct((n, 1), f32),
            jax.ShapeDtypeStruct((n, 3), f32),
        ],
    )(sigma2, tex0, tex1, tex2, sampled_depth, sampled_dists, sampled_idx)


def kernel(ray_start, ray_dir, sampled_depth, sampled_idx, sampled_dists, emb,
           W1, b1, W2, b2, Wsig, bsig, Wt1, bt1, Wt2, bt2):
    n, k = sampled_depth.shape
    nk = n * k
    hid = W2.shape[0]

    idx_flat = jnp.maximum(sampled_idx.reshape(nk), 0).astype(jnp.int32)
    v, de0 = emb.shape
    per = 256 // de0
    unit = _SC_SUBCORES * 56
    vq = -(-(v // per) // unit) * unit
    bf = jnp.bfloat16
    embp = jnp.pad(emb.astype(bf), ((0, vq * per - v), (0, 0))).reshape(vq, 256)
    lo = lax.bitcast_convert_type(embp[:, :128], jnp.uint16).astype(jnp.uint32)
    hi = lax.bitcast_convert_type(embp[:, 128:], jnp.uint16).astype(jnp.uint32)
    emb3 = lax.bitcast_convert_type(lo | (hi << 16), jnp.int32)
    g_flat = idx_flat // per
    q_flat = (idx_flat % per).astype(jnp.float32).reshape(nk, 1)

    rs_flat = jnp.broadcast_to(ray_start[:, None, :], (n, k, 3)).reshape(nk, 3)
    rd_flat = jnp.broadcast_to(ray_dir[:, None, :], (n, k, 3)).reshape(nk, 3)
    dep_flat = sampled_depth.reshape(nk, 1)
    geom = jnp.concatenate([rs_flat, rd_flat, dep_flat, q_flat], axis=1)

    w1b = W1[:39].astype(bf)
    w1c4 = jnp.concatenate([W1[39:]] * 4, axis=0).astype(bf)
    b1r, b2r = b1.reshape(1, -1), b2.reshape(1, -1)
    bsigr, bt1r, bt2r = bsig.reshape(1, 1), bt1.reshape(1, -1), bt2.reshape(1, -1)
    w2b, wsigb, wt2b = W2.astype(bf), Wsig.astype(bf), Wt2.astype(bf)
    wt1hb, wt1db = Wt1[:hid].astype(bf), Wt1[hid:].astype(bf)

    n_stripe = 8
    stripe = nk // n_stripe
    outs = []
    for s in range(n_stripe):
        sl = slice(s * stripe, (s + 1) * stripe)
        feat_s = _sc_gather(emb3, g_flat[sl])
        outs.append(_mlp_call(
            geom[sl], feat_s,
            w1b, w1c4, b1r, w2b, b2r, wsigb, bsigr,
            wt1hb, wt1db, bt1r, wt2b, bt2r, p_blk=4096))
    st = jnp.concatenate(outs, axis=0).reshape(n, k, 4)

    probs, depths, missed, colors = _render_call(
        st[..., 0], st[..., 1], st[..., 2], st[..., 3],
        sampled_depth, sampled_dists, sampled_idx, r_blk=512)
    return probs, depths.reshape(n), missed.reshape(n), colors
